# baseline (speedup 1.0000x reference)
import jax
import jax.numpy as jnp
from jax.experimental import pallas as pl
from jax.experimental.pallas import tpu as pltpu

_START, _END = 512, 2560
_ZROWS = 512


def _dma_kernel(g_ref, o_ref, zbuf, sems):
    b0s, b1s, g_rows, feat = g_ref.shape
    seq = o_ref.shape[2]
    copies = []
    i = 0
    for b0 in range(b0s):
        for b1 in range(b1s):
            c = pltpu.make_async_copy(
                g_ref.at[b0, b1],
                o_ref.at[b0, b1, pl.ds(_START, g_rows)],
                sems.at[i],
            )
            c.start()
            copies.append(c)
            i += 1
    zbuf[...] = jnp.zeros_like(zbuf)
    for b0 in range(b0s):
        for b1 in range(b1s):
            for r0 in range(0, _START, _ZROWS):
                c = pltpu.make_async_copy(
                    zbuf, o_ref.at[b0, b1, pl.ds(r0, _ZROWS)], sems.at[i]
                )
                c.start()
                copies.append(c)
                i += 1
            for r0 in range(_END, seq, _ZROWS):
                c = pltpu.make_async_copy(
                    zbuf, o_ref.at[b0, b1, pl.ds(r0, _ZROWS)], sems.at[i]
                )
                c.start()
                copies.append(c)
                i += 1
    for c in copies:
        c.wait()


def kernel(grad_last, input):
    b0s, b1s, g_rows, feat = grad_last.shape
    seq = input.shape[1]
    n_zero = (_START // _ZROWS + (seq - _END) // _ZROWS) * b0s * b1s
    n_dma = b0s * b1s + n_zero
    return pl.pallas_call(
        _dma_kernel,
        in_specs=[pl.BlockSpec(memory_space=pltpu.MemorySpace.HBM)],
        out_specs=pl.BlockSpec(memory_space=pltpu.MemorySpace.HBM),
        out_shape=jax.ShapeDtypeStruct((b0s, b1s, seq, feat), grad_last.dtype),
        scratch_shapes=[
            pltpu.VMEM((_ZROWS, feat), grad_last.dtype),
            pltpu.SemaphoreType.DMA((n_dma,)),
        ],
    )(grad_last)


# speedup vs baseline: 33.9446x; 33.9446x over previous
import jax
import jax.numpy as jnp
from jax.experimental import pallas as pl
from jax.experimental.pallas import tpu as pltpu

_START, _END = 512, 2560


def _pad_kernel(g_ref, o_ref):
    o_ref[0, : _START, :] = jnp.zeros_like(o_ref[0, : _START, :])
    o_ref[0, _START:_END, :] = g_ref[0]
    o_ref[0, _END:, :] = jnp.zeros_like(o_ref[0, _END:, :])


def kernel(grad_last, input):
    b0, b1, g_rows, feat = grad_last.shape
    seq = input.shape[1]
    nb = b0 * b1
    g = grad_last.reshape(nb, g_rows, feat)

    out = pl.pallas_call(
        _pad_kernel,
        grid=(nb,),
        in_specs=[pl.BlockSpec((1, g_rows, feat), lambda b: (b, 0, 0))],
        out_specs=pl.BlockSpec((1, seq, feat), lambda b: (b, 0, 0)),
        out_shape=jax.ShapeDtypeStruct((nb, seq, feat), grad_last.dtype),
        compiler_params=pltpu.CompilerParams(
            dimension_semantics=("parallel",),
        ),
    )(g)
    return out.reshape(b0, b1, seq, feat)
